# baseline (speedup 1.0000x reference)
import functools

import jax
import jax.numpy as jnp
from jax import lax
from jax.experimental import pallas as pl
from jax.experimental.pallas import tpu as pltpu
from jax.experimental.pallas import tpu_sc as plsc

NUM_USERS = 1000000
NUM_ITEMS = 100000
DIM = 64
BATCH = 16384

NC = 2
NS = 16
NW = NC * NS
B_PER_W = BATCH // NW
CHUNK = 128
NCHUNK = B_PER_W // CHUNK
GROUPS = B_PER_W // 16
LAM_W = 16


def _log_sigmoid(x):
    t = jnp.exp(-jnp.abs(x))
    w = t / (t + 2.0)
    w2 = w * w
    poly = 1.0 + w2 * (1.0 / 3.0 + w2 * (1.0 / 5.0 + w2 * (1.0 / 7.0 + w2 * (1.0 / 9.0))))
    return jnp.minimum(x, 0.0) - 2.0 * w * poly


def _sc_body(uidx_hbm, iidx_hbm, theta_hbm, alpha_hbm, lam_hbm, out_hbm,
             idx_u, idx_i, idx_hi, u_rows, a_rows, lam_rows, out_buf,
             sem0, sem1, sem2, sem3):
    c = lax.axis_index("c")
    s = lax.axis_index("s")
    wid = s * NC + c
    sems = [sem0, sem1, sem2, sem3]

    pltpu.sync_copy(uidx_hbm.at[wid], idx_u)
    pltpu.sync_copy(iidx_hbm.at[wid], idx_i)

    lane = lax.iota(jnp.int32, 16)

    copies = [[] for _ in range(NCHUNK)]
    for j in range(NCHUNK):
        dst = pl.ds(j * CHUNK, CHUNK)
        copies[j].append(pltpu.async_copy(theta_hbm.at[idx_u.at[j]], u_rows.at[dst], sems[j]))
        copies[j].append(pltpu.async_copy(alpha_hbm.at[idx_i.at[j]], a_rows.at[dst], sems[j]))

    def hi_step(k, _):
        ch = jnp.full((16,), k >> 3, jnp.int32)
        pos = jnp.full((16,), (k & 7) * 16, jnp.int32) + lane
        iv = plsc.load_gather(idx_i, [ch, pos])
        plsc.store_scatter(idx_hi, [ch, pos], iv >> 4)
        return 0

    lax.fori_loop(0, GROUPS, hi_step, 0)

    for j in range(NCHUNK):
        dst = pl.ds(j * CHUNK, CHUNK)
        copies[j].append(pltpu.async_copy(lam_hbm.at[idx_hi.at[j]], lam_rows.at[dst], sems[j]))

    zero = jnp.zeros((16,), jnp.float32)

    def group(g, _):
        ch = jnp.full((16,), g >> 3, jnp.int32)
        pos = jnp.full((16,), (g & 7) * 16, jnp.int32) + lane
        row_ids = jnp.full((16,), g * 16, jnp.int32) + lane

        def dstep(t, carry):
            a0, a1, a2, a3, dv = carry
            accs = [a0, a1, a2, a3]
            for k in range(8):
                dk = dv + k if k else dv
                uv = plsc.load_gather(u_rows, [row_ids, dk])
                av = plsc.load_gather(a_rows, [row_ids, dk])
                accs[k & 3] = accs[k & 3] + uv * av
            return (accs[0], accs[1], accs[2], accs[3], dv + 8)

        a0, a1, a2, a3, _ = lax.fori_loop(
            0, DIM // 8, dstep, (zero, zero, zero, zero, jnp.zeros((16,), jnp.int32)))
        acc = (a0 + a1) + (a2 + a3)
        iv = plsc.load_gather(idx_i, [ch, pos])
        lamv = plsc.load_gather(lam_rows, [row_ids, iv & 15])
        out_buf[pl.ds(g * 16, 16)] = _log_sigmoid(acc + lamv)
        return 0

    for j in range(NCHUNK):
        for cp in copies[j]:
            cp.wait()
        lax.fori_loop(j * (CHUNK // 16), (j + 1) * (CHUNK // 16), group, 0)

    pltpu.sync_copy(out_buf, out_hbm.at[pl.ds(wid * B_PER_W, B_PER_W)])


@jax.jit
def _run(uidx, iidx, theta_user, alpha_item, lam2d):
    mesh = plsc.VectorSubcoreMesh(core_axis_name="c", subcore_axis_name="s")
    f = functools.partial(
        pl.kernel,
        mesh=mesh,
        out_type=jax.ShapeDtypeStruct((BATCH,), jnp.float32),
        compiler_params=pltpu.CompilerParams(
            needs_layout_passes=False, use_tc_tiling_on_sc=False),
        scratch_types=[
            pltpu.VMEM((NCHUNK, CHUNK), jnp.int32),
            pltpu.VMEM((NCHUNK, CHUNK), jnp.int32),
            pltpu.VMEM((NCHUNK, CHUNK), jnp.int32),
            pltpu.VMEM((B_PER_W, DIM), jnp.float32),
            pltpu.VMEM((B_PER_W, DIM), jnp.float32),
            pltpu.VMEM((B_PER_W, LAM_W), jnp.float32),
            pltpu.VMEM((B_PER_W,), jnp.float32),
            pltpu.SemaphoreType.DMA,
            pltpu.SemaphoreType.DMA,
            pltpu.SemaphoreType.DMA,
            pltpu.SemaphoreType.DMA,
        ],
    )(_sc_body)
    return f(uidx, iidx, theta_user, alpha_item, lam2d)


def kernel(user_index, item_index, theta_user, alpha_item, lambda_item):
    uidx = user_index.astype(jnp.int32).reshape(NW, NCHUNK, CHUNK)
    iidx = item_index.astype(jnp.int32).reshape(NW, NCHUNK, CHUNK)
    lam2d = lambda_item.reshape(NUM_ITEMS // LAM_W, LAM_W)
    return _run(uidx, iidx, theta_user, alpha_item, lam2d)
